# Initial kernel scaffold; baseline (speedup 1.0000x reference)
import jax
import jax.numpy as jnp
from jax.experimental import pallas as pl


def kernel(inputs, src_emb, tgt_emb):
    raise NotImplementedError("write your pallas kernel here")


# speedup vs baseline: 1.3562x; 1.3562x over previous
import functools

import jax
import jax.numpy as jnp
from jax import lax
from jax.experimental import pallas as pl
from jax.experimental.pallas import tpu as pltpu
from jax.experimental.pallas import tpu_sc as plsc

MAX_LEN = 50
BATCH = 16384
EMBED = 32
TOTAL = MAX_LEN * BATCH
NUM_WORKERS = 32
PER_WORKER = TOTAL // NUM_WORKERS
CHUNK = 1024
NCHUNK = PER_WORKER // CHUNK

_mesh = plsc.VectorSubcoreMesh(core_axis_name="c", subcore_axis_name="s")


@functools.partial(
    pl.kernel,
    out_type=jax.ShapeDtypeStruct((TOTAL, EMBED), jnp.float32),
    mesh=_mesh,
    compiler_params=pltpu.CompilerParams(use_tc_tiling_on_sc=False),
    scratch_types=[
        pltpu.VMEM((PER_WORKER,), jnp.int32),
        pltpu.VMEM((CHUNK, EMBED), jnp.float32),
        pltpu.SemaphoreType.DMA,
    ],
)
def _embedding_gather(idx_hbm, table_hbm, out_hbm, idx_v, rows_v, gsem):
    wid = lax.axis_index("s") * 2 + lax.axis_index("c")
    base = wid * PER_WORKER
    pltpu.sync_copy(idx_hbm.at[pl.ds(base, PER_WORKER)], idx_v)

    def body(g, carry):
        off = g * CHUNK
        pltpu.async_copy(
            table_hbm.at[idx_v.at[pl.ds(off, CHUNK)]], rows_v, gsem
        ).wait()
        pltpu.sync_copy(rows_v, out_hbm.at[pl.ds(base + off, CHUNK)])
        return carry

    lax.fori_loop(0, NCHUNK, body, 0)


def kernel(inputs, src_emb, tgt_emb):
    del tgt_emb
    flat_idx = inputs.reshape(TOTAL)
    out = _embedding_gather(flat_idx, src_emb)
    return out.reshape(MAX_LEN, BATCH, EMBED)


# speedup vs baseline: 1.3714x; 1.0112x over previous
import functools

import jax
import jax.numpy as jnp
from jax import lax
from jax.experimental import pallas as pl
from jax.experimental.pallas import tpu as pltpu
from jax.experimental.pallas import tpu_sc as plsc

MAX_LEN = 50
BATCH = 16384
EMBED = 32
TOTAL = MAX_LEN * BATCH
NUM_WORKERS = 32
PER_WORKER = TOTAL // NUM_WORKERS
CHUNK = 640
NCHUNK = PER_WORKER // CHUNK
NBUF = 4
NGROUP = NCHUNK // NBUF

_mesh = plsc.VectorSubcoreMesh(core_axis_name="c", subcore_axis_name="s")


@functools.partial(
    pl.kernel,
    out_type=jax.ShapeDtypeStruct((TOTAL, EMBED), jnp.float32),
    mesh=_mesh,
    compiler_params=pltpu.CompilerParams(use_tc_tiling_on_sc=False),
    scratch_types=[
        pltpu.VMEM((PER_WORKER,), jnp.int32),
        pltpu.VMEM((NBUF, CHUNK, EMBED), jnp.float32),
        [pltpu.SemaphoreType.DMA] * NBUF,
        [pltpu.SemaphoreType.DMA] * NBUF,
    ],
)
def _embedding_gather(idx_hbm, table_hbm, out_hbm, idx_v, rows_v, gsems, ssems):
    wid = lax.axis_index("s") * 2 + lax.axis_index("c")
    base = wid * PER_WORKER
    pltpu.sync_copy(idx_hbm.at[pl.ds(base, PER_WORKER)], idx_v)

    def fire_gather(g, b):
        pltpu.async_copy(
            table_hbm.at[idx_v.at[pl.ds(g * CHUNK, CHUNK)]],
            rows_v.at[b], gsems[b])

    def wait_gather(b):
        pltpu.make_async_copy(
            table_hbm.at[idx_v.at[pl.ds(0, CHUNK)]],
            rows_v.at[b], gsems[b]).wait()

    def fire_store(g, b):
        pltpu.async_copy(
            rows_v.at[b], out_hbm.at[pl.ds(base + g * CHUNK, CHUNK)], ssems[b])

    def wait_store(b):
        pltpu.make_async_copy(
            rows_v.at[b], out_hbm.at[pl.ds(base, CHUNK)], ssems[b]).wait()

    for b in range(NBUF):
        fire_gather(b, b)

    def body(go, carry):
        for b in range(NBUF):
            wait_gather(b)
            fire_store(go * NBUF + b, b)
        for b in range(NBUF):
            wait_store(b)
            fire_gather((go + 1) * NBUF + b, b)
        return carry

    lax.fori_loop(0, NGROUP - 1, body, 0)

    last = (NGROUP - 1) * NBUF
    for b in range(NBUF):
        wait_gather(b)
        fire_store(last + b, b)
    for b in range(NBUF):
        wait_store(b)


def kernel(inputs, src_emb, tgt_emb):
    del tgt_emb
    flat_idx = inputs.reshape(TOTAL)
    out = _embedding_gather(flat_idx, src_emb)
    return out.reshape(MAX_LEN, BATCH, EMBED)
